# baseline (speedup 1.0000x reference)
import functools

import jax
import jax.numpy as jnp
import numpy as np
from jax import lax
from jax.experimental import pallas as pl
from jax.experimental.pallas import tpu as pltpu
from jax.experimental.pallas import tpu_sc as plsc

N_LEVELS = 16
F = 2
LOG2_T = 19
T = 2 ** LOG2_T
RES = [16, 24, 36, 54, 81, 121, 182, 273, 410, 615, 922, 1383,
       2075, 3113, 4670, 7006]
P1 = np.uint32(2654435761)
P2 = np.uint32(805459861)
MASK = np.uint32(T - 1)

NW = 32
C = 1024
NV = C // 16
D_OUT = 3 + N_LEVELS * F


def _corners():
    out = []
    for cx in range(2):
        for cy in range(2):
            for cz in range(2):
                out.append((cx, cy, cz))
    return out


def _body(n_points, xt, tbl, out, xv, yv, zv, oxv, oyv, ozv,
          wxv, wyv, wzv, idxv, rowsv, outv, sem):
    npw = n_points // NW
    nch = npw // C
    wid = lax.axis_index("s") * 2 + lax.axis_index("c")
    base = wid * npw
    iota = lax.iota(jnp.int32, 16)

    def chunk_body(ch, _):
        p0 = base + ch * C
        pltpu.sync_copy(xt.at[0, pl.ds(p0, C)], oxv)
        pltpu.sync_copy(xt.at[1, pl.ds(p0, C)], oyv)
        pltpu.sync_copy(xt.at[2, pl.ds(p0, C)], ozv)
        pltpu.sync_copy(xt.at[3, pl.ds(p0, C)], xv)
        pltpu.sync_copy(xt.at[4, pl.ds(p0, C)], yv)
        pltpu.sync_copy(xt.at[5, pl.ds(p0, C)], zv)

        def xyz_body(v, _):
            sl = pl.ds(v * 16, 16)
            outv[0, sl] = oxv[sl]
            outv[1, sl] = oyv[sl]
            outv[2, sl] = ozv[sl]
            return 0
        lax.fori_loop(0, NV, xyz_body, 0)

        for l in range(N_LEVELS):
            res = float(RES[l])
            lshift = np.uint32(l * T)

            def idx_body(v, _, res=res, lshift=lshift):
                sl = pl.ds(v * 16, 16)
                px = xv[sl] * res
                py = yv[sl] * res
                pz = zv[sl] * res
                ix = px.astype(jnp.int32)
                iy = py.astype(jnp.int32)
                iz = pz.astype(jnp.int32)
                wxv[sl] = px - ix.astype(jnp.float32)
                wyv[sl] = py - iy.astype(jnp.float32)
                wzv[sl] = pz - iz.astype(jnp.float32)
                hx0 = ix.astype(jnp.uint32)
                hy0 = iy.astype(jnp.uint32) * P1
                hz0 = iz.astype(jnp.uint32) * P2
                hx = (hx0, hx0 + jnp.uint32(1))
                hy = (hy0, hy0 + P1)
                hz = (hz0, hz0 + P2)
                for c, (cx, cy, cz) in enumerate(_corners()):
                    h = ((hx[cx] ^ hy[cy] ^ hz[cz]) & MASK) | lshift
                    idxv[pl.ds(v * 128 + c * 16, 16)] = h.astype(jnp.int32)
                return 0
            lax.fori_loop(0, NV, idx_body, 0)

            cp = pltpu.make_async_copy(tbl.at[idxv], rowsv, sem)
            cp.start()
            cp.wait()

            def acc_body(v, _, l=l):
                sl = pl.ds(v * 16, 16)
                wx = wxv[sl]
                wy = wyv[sl]
                wz = wzv[sl]
                cwx = (1.0 - wx, wx)
                cwy = (1.0 - wy, wy)
                cwz = (1.0 - wz, wz)
                f0 = jnp.zeros((16,), jnp.float32)
                f1 = jnp.zeros((16,), jnp.float32)
                zc = jnp.zeros((16,), jnp.int32)
                oc = jnp.ones((16,), jnp.int32)
                for c, (cx, cy, cz) in enumerate(_corners()):
                    wt = cwx[cx] * cwy[cy] * cwz[cz]
                    col = v * 128 + c * 16 + iota
                    g0 = plsc.load_gather(rowsv, [col, zc])
                    g1 = plsc.load_gather(rowsv, [col, oc])
                    f0 = f0 + wt * g0
                    f1 = f1 + wt * g1
                sl16 = pl.ds(v * 16, 16)
                outv[3 + 2 * l, sl16] = f0
                outv[4 + 2 * l, sl16] = f1
                return 0
            lax.fori_loop(0, NV, acc_body, 0)

        pltpu.sync_copy(outv, out.at[:, pl.ds(p0, C)])
        return 0

    lax.fori_loop(0, nch, chunk_body, 0)


def kernel(input, bound_min, bound_max, table):
    n = input.shape[0]
    xn = (input - bound_min[None, :]) / (bound_max - bound_min)[None, :]
    xt = jnp.concatenate([input.T, xn.T], axis=0)
    tbl = table.reshape(N_LEVELS * T, F)

    mesh = plsc.VectorSubcoreMesh(core_axis_name="c", subcore_axis_name="s")
    run = pl.kernel(
        functools.partial(_body, n),
        out_type=jax.ShapeDtypeStruct((D_OUT, n), jnp.float32),
        mesh=mesh,
        scratch_types=[
            pltpu.VMEM((C,), jnp.float32),
            pltpu.VMEM((C,), jnp.float32),
            pltpu.VMEM((C,), jnp.float32),
            pltpu.VMEM((C,), jnp.float32),
            pltpu.VMEM((C,), jnp.float32),
            pltpu.VMEM((C,), jnp.float32),
            pltpu.VMEM((C,), jnp.float32),
            pltpu.VMEM((C,), jnp.float32),
            pltpu.VMEM((C,), jnp.float32),
            pltpu.VMEM((8 * C,), jnp.int32),
            pltpu.VMEM((8 * C, F), jnp.float32),
            pltpu.VMEM((D_OUT, C), jnp.float32),
            pltpu.SemaphoreType.DMA,
        ],
        compiler_params=pltpu.CompilerParams(
            needs_layout_passes=False, use_tc_tiling_on_sc=False),
    )
    return run(xt, tbl).T


# speedup vs baseline: 1.0193x; 1.0193x over previous
import functools

import jax
import jax.numpy as jnp
import numpy as np
from jax import lax
from jax.experimental import pallas as pl
from jax.experimental.pallas import tpu as pltpu
from jax.experimental.pallas import tpu_sc as plsc

N_LEVELS = 16
F = 2
LOG2_T = 19
T = 2 ** LOG2_T
RES = [16, 24, 36, 54, 81, 121, 182, 273, 410, 615, 922, 1383,
       2075, 3113, 4670, 7006]
P1 = np.uint32(2654435761)
P2 = np.uint32(805459861)
MASK = np.uint32(T - 1)

NW = 32
C = 512
NV = C // 16
D_OUT = 3 + N_LEVELS * F
KSUB = 8
B2 = 8 * C // KSUB
NSB = C // 16

_LANES = np.arange(16)
_CTAB = np.stack([(j * 16 + _LANES) % D_OUT for j in range(D_OUT)])
_PTAB = np.stack([(j * 16 + _LANES) // D_OUT for j in range(D_OUT)])
_RELAYOUT_TAB = np.concatenate(
    [_CTAB.reshape(-1), _PTAB.reshape(-1)]).astype(np.int32)


def _corners():
    return [(cx, cy, cz) for cx in range(2) for cy in range(2)
            for cz in range(2)]


def _body(n_points, xt, tbl, rtab, out, rtabv, xv, yv, zv, oxv, oyv, ozv,
          wx0, wy0, wz0, wx1, wy1, wz1, idx0, idx1, rows0, rows1,
          outv, outiv, sem0, sem1):
    npw = n_points // NW
    nch = npw // C
    wid = lax.axis_index("s") * 2 + lax.axis_index("c")
    base = wid * npw
    iota = lax.iota(jnp.int32, 16)
    wbufs = [(wx0, wy0, wz0), (wx1, wy1, wz1)]
    idxbufs = [idx0, idx1]
    rowbufs = [rows0, rows1]
    sems = [sem0, sem1]

    pltpu.sync_copy(rtab, rtabv)

    def idx_pass(l):
        res = float(RES[l])
        lshift = np.uint32(l * T)
        wxv, wyv, wzv = wbufs[l % 2]
        idxv = idxbufs[l % 2]

        @plsc.parallel_loop(0, NV)
        def idx_body(v):
            sl = pl.ds(v * 16, 16)
            px = xv[sl] * res
            py = yv[sl] * res
            pz = zv[sl] * res
            ix = px.astype(jnp.int32)
            iy = py.astype(jnp.int32)
            iz = pz.astype(jnp.int32)
            wxv[sl] = px - ix.astype(jnp.float32)
            wyv[sl] = py - iy.astype(jnp.float32)
            wzv[sl] = pz - iz.astype(jnp.float32)
            hx0 = ix.astype(jnp.uint32)
            hy0 = iy.astype(jnp.uint32) * P1
            hz0 = iz.astype(jnp.uint32) * P2
            hx = (hx0, hx0 + jnp.uint32(1))
            hy = (hy0, hy0 + P1)
            hz = (hz0, hz0 + P2)
            for c, (cx, cy, cz) in enumerate(_corners()):
                h = ((hx[cx] ^ hy[cy] ^ hz[cz]) & MASK) | lshift
                idxv[pl.ds(v * 128 + c * 16, 16)] = h.astype(jnp.int32)

    def acc_pass(l):
        wxv, wyv, wzv = wbufs[l % 2]
        rowsv = rowbufs[l % 2]

        @plsc.parallel_loop(0, NV)
        def acc_body(v):
            sl = pl.ds(v * 16, 16)
            wx = wxv[sl]
            wy = wyv[sl]
            wz = wzv[sl]
            cwx = (1.0 - wx, wx)
            cwy = (1.0 - wy, wy)
            cwz = (1.0 - wz, wz)
            zc = jnp.zeros((16,), jnp.int32)
            oc = jnp.ones((16,), jnp.int32)
            t0 = []
            t1 = []
            for c, (cx, cy, cz) in enumerate(_corners()):
                wt = cwx[cx] * cwy[cy] * cwz[cz]
                col = v * 128 + c * 16 + iota
                g0 = plsc.load_gather(rowsv, [col, zc])
                g1 = plsc.load_gather(rowsv, [col, oc])
                t0.append(wt * g0)
                t1.append(wt * g1)
            while len(t0) > 1:
                t0 = [a + b for a, b in zip(t0[::2], t0[1::2])]
                t1 = [a + b for a, b in zip(t1[::2], t1[1::2])]
            outv[3 + 2 * l, sl] = t0[0]
            outv[4 + 2 * l, sl] = t1[0]

    def gather_start(l):
        idxv = idxbufs[l % 2]
        rowsv = rowbufs[l % 2]
        sem = sems[l % 2]
        for j in range(KSUB):
            sl = pl.ds(j * B2, B2)
            pltpu.make_async_copy(tbl.at[idxv.at[sl]], rowsv.at[sl, :],
                                  sem).start()

    def gather_drain(l):
        idxv = idxbufs[l % 2]
        rowsv = rowbufs[l % 2]
        sem = sems[l % 2]
        for j in range(KSUB):
            sl = pl.ds(j * B2, B2)
            pltpu.make_async_copy(tbl.at[idxv.at[sl]], rowsv.at[sl, :],
                                  sem).wait()

    def chunk_body(ch, _):
        p0 = base + ch * C
        pltpu.sync_copy(xt.at[0, pl.ds(p0, C)], oxv)
        pltpu.sync_copy(xt.at[1, pl.ds(p0, C)], oyv)
        pltpu.sync_copy(xt.at[2, pl.ds(p0, C)], ozv)
        pltpu.sync_copy(xt.at[3, pl.ds(p0, C)], xv)
        pltpu.sync_copy(xt.at[4, pl.ds(p0, C)], yv)
        pltpu.sync_copy(xt.at[5, pl.ds(p0, C)], zv)

        @plsc.parallel_loop(0, NV)
        def xyz_body(v):
            sl = pl.ds(v * 16, 16)
            outv[0, sl] = oxv[sl]
            outv[1, sl] = oyv[sl]
            outv[2, sl] = ozv[sl]

        idx_pass(0)
        gather_start(0)
        for l in range(1, N_LEVELS):
            idx_pass(l)
            gather_start(l)
            gather_drain(l - 1)
            acc_pass(l - 1)
        gather_drain(N_LEVELS - 1)
        acc_pass(N_LEVELS - 1)

        @plsc.parallel_loop(0, NSB)
        def relayout_body(s):
            pbase = s * 16
            for j in range(D_OUT):
                colv = rtabv[pl.ds(j * 16, 16)]
                pointv = rtabv[pl.ds(D_OUT * 16 + j * 16, 16)] + pbase
                g = plsc.load_gather(outv, [colv, pointv])
                outiv[pl.ds(s * (16 * D_OUT) + j * 16, 16)] = g

        pltpu.sync_copy(outiv, out.at[pl.ds(p0 * D_OUT, C * D_OUT)])
        return 0

    lax.fori_loop(0, nch, chunk_body, 0)


def kernel(input, bound_min, bound_max, table):
    n = input.shape[0]
    xn = (input - bound_min[None, :]) / (bound_max - bound_min)[None, :]
    xt = jnp.concatenate([input.T, xn.T], axis=0)
    tbl = table.reshape(N_LEVELS * T, F)
    rtab = jnp.asarray(_RELAYOUT_TAB)

    mesh = plsc.VectorSubcoreMesh(core_axis_name="c", subcore_axis_name="s")
    run = pl.kernel(
        functools.partial(_body, n),
        out_type=jax.ShapeDtypeStruct((n * D_OUT,), jnp.float32),
        mesh=mesh,
        scratch_types=[
            pltpu.VMEM((2 * D_OUT * 16,), jnp.int32),
            pltpu.VMEM((C,), jnp.float32),
            pltpu.VMEM((C,), jnp.float32),
            pltpu.VMEM((C,), jnp.float32),
            pltpu.VMEM((C,), jnp.float32),
            pltpu.VMEM((C,), jnp.float32),
            pltpu.VMEM((C,), jnp.float32),
            pltpu.VMEM((C,), jnp.float32),
            pltpu.VMEM((C,), jnp.float32),
            pltpu.VMEM((C,), jnp.float32),
            pltpu.VMEM((C,), jnp.float32),
            pltpu.VMEM((C,), jnp.float32),
            pltpu.VMEM((C,), jnp.float32),
            pltpu.VMEM((8 * C,), jnp.int32),
            pltpu.VMEM((8 * C,), jnp.int32),
            pltpu.VMEM((8 * C, F), jnp.float32),
            pltpu.VMEM((8 * C, F), jnp.float32),
            pltpu.VMEM((D_OUT, C), jnp.float32),
            pltpu.VMEM((C * D_OUT,), jnp.float32),
            pltpu.SemaphoreType.DMA,
            pltpu.SemaphoreType.DMA,
        ],
        compiler_params=pltpu.CompilerParams(
            needs_layout_passes=False, use_tc_tiling_on_sc=False),
    )
    return run(xt, tbl, rtab).reshape(n, D_OUT)


# speedup vs baseline: 1.0257x; 1.0064x over previous
import functools

import jax
import jax.numpy as jnp
import numpy as np
from jax import lax
from jax.experimental import pallas as pl
from jax.experimental.pallas import tpu as pltpu
from jax.experimental.pallas import tpu_sc as plsc

N_LEVELS = 16
F = 2
LOG2_T = 19
T = 2 ** LOG2_T
RES = [16, 24, 36, 54, 81, 121, 182, 273, 410, 615, 922, 1383,
       2075, 3113, 4670, 7006]
P1 = np.uint32(2654435761)
P2 = np.uint32(805459861)
MASK = np.uint32(T - 1)

NW = 32
C = 256
NV = C // 16
D_OUT = 3 + N_LEVELS * F
KSUB = 8
B2 = 8 * C // KSUB
NSB = C // 16

_LANES = np.arange(16)
_CTAB = np.stack([(j * 16 + _LANES) % D_OUT for j in range(D_OUT)])
_PTAB = np.stack([(j * 16 + _LANES) // D_OUT for j in range(D_OUT)])
_RELAYOUT_TAB = np.concatenate(
    [_CTAB.reshape(-1), _PTAB.reshape(-1)]).astype(np.int32)


def _corners():
    return [(cx, cy, cz) for cx in range(2) for cy in range(2)
            for cz in range(2)]


LUT_LEVELS = [0, 1]
LUT_S = [RES[l] + 1 for l in LUT_LEVELS]
LUT_K = [s3 ** 3 for s3 in LUT_S]
BB = 2048


def _lut_pad(k):
    nb = (k + BB - 1) // BB
    last = k - (nb - 1) * BB
    return (nb - 1) * BB + ((last + 15) // 16) * 16


def _dense_hash_tab():
    parts = []
    for li, l in enumerate(LUT_LEVELS):
        S = LUT_S[li]
        d = np.arange(_lut_pad(LUT_K[li]), dtype=np.int64)
        x = (d % S).astype(np.uint32)
        y = ((d // S) % S).astype(np.uint32)
        z = ((d // (S * S)) % S).astype(np.uint32)
        h = (x * np.uint32(1)) ^ (y * P1) ^ (z * P2)
        parts.append(((h & MASK) | np.uint32(l * T)).astype(np.int32))
    return np.concatenate(parts)


_DHTAB = _dense_hash_tab()
_DH_OFF = [0, _lut_pad(LUT_K[0])]


def _body(n_points, in_flat, prm, tbl, rtab, dhtab, out, rtabv, prmv, inbuf,
          xv, yv, zv, wx0, wy0, wz0, wx1, wy1, wz1, idx0, idx1,
          rows0, rows1, outv, outiv, l0f0, l0f1, l1f0, l1f1, sem0, sem1):
    npw = n_points // NW
    nch = npw // C
    wid = lax.axis_index("s") * 2 + lax.axis_index("c")
    base = wid * npw
    iota = lax.iota(jnp.int32, 16)
    wbufs = [(wx0, wy0, wz0), (wx1, wy1, wz1)]
    idxbufs = [idx0, idx1]
    rowbufs = [rows0, rows1]
    sems = [sem0, sem1]

    pltpu.sync_copy(rtab, rtabv)
    pltpu.sync_copy(prm, prmv)
    off0 = prmv[pl.ds(0, 16)]
    off1 = prmv[pl.ds(16, 16)]
    off2 = prmv[pl.ds(32, 16)]
    sc0 = prmv[pl.ds(48, 16)]
    sc1 = prmv[pl.ds(64, 16)]
    sc2 = prmv[pl.ds(80, 16)]

    zc16 = jnp.zeros((16,), jnp.int32)
    oc16 = jnp.ones((16,), jnp.int32)

    for li, l in enumerate(LUT_LEVELS):
        S = LUT_S[li]
        K = LUT_K[li]
        lshift = np.uint32(l * T)
        planes = (l0f0, l0f1) if li == 0 else (l1f0, l1f1)
        nb = (K + BB - 1) // BB
        for b in range(nb):
            nv_b = (min(BB, K - b * BB) + 15) // 16
            pltpu.sync_copy(
                dhtab.at[pl.ds(_DH_OFF[li] + b * BB, nv_b * 16)],
                idx0.at[pl.ds(0, nv_b * 16)])

            nsub = (nv_b * 16 + B2 - 1) // B2
            for j in range(nsub):
                sl = pl.ds(j * B2, min(B2, nv_b * 16 - j * B2))
                pltpu.make_async_copy(tbl.at[idx0.at[sl]], rows0.at[sl, :],
                                      sem0).start()
            for j in range(nsub):
                sl = pl.ds(j * B2, min(B2, nv_b * 16 - j * B2))
                pltpu.make_async_copy(tbl.at[idx0.at[sl]], rows0.at[sl, :],
                                      sem0).wait()

            @plsc.parallel_loop(0, nv_b)
            def bcompact_body(v, b=b, planes=planes):
                col = v * 16 + iota
                g0 = plsc.load_gather(rows0, [col, zc16])
                g1 = plsc.load_gather(rows0, [col, oc16])
                planes[0][pl.ds(b * BB + v * 16, 16)] = g0
                planes[1][pl.ds(b * BB + v * 16, 16)] = g1

    def lut_pass(li):
        l = LUT_LEVELS[li]
        res = float(RES[l])
        S = LUT_S[li]
        planes = (l0f0, l0f1) if li == 0 else (l1f0, l1f1)

        @plsc.parallel_loop(0, NV)
        def lut_body(v):
            sl = pl.ds(v * 16, 16)
            px = xv[sl] * res
            py = yv[sl] * res
            pz = zv[sl] * res
            ix = px.astype(jnp.int32)
            iy = py.astype(jnp.int32)
            iz = pz.astype(jnp.int32)
            wx = px - ix.astype(jnp.float32)
            wy = py - iy.astype(jnp.float32)
            wz = pz - iz.astype(jnp.float32)
            cwx = (1.0 - wx, wx)
            cwy = (1.0 - wy, wy)
            cwz = (1.0 - wz, wz)
            a = ix + S * iy + (S * S) * iz
            t0 = []
            t1 = []
            for cx, cy, cz in _corners():
                wt = cwx[cx] * cwy[cy] * cwz[cz]
                d = a + (cx + S * cy + (S * S) * cz)
                g0 = plsc.load_gather(planes[0], [d])
                g1 = plsc.load_gather(planes[1], [d])
                t0.append(wt * g0)
                t1.append(wt * g1)
            while len(t0) > 1:
                t0 = [p + q for p, q in zip(t0[::2], t0[1::2])]
                t1 = [p + q for p, q in zip(t1[::2], t1[1::2])]
            outv[3 + 2 * l, sl] = t0[0]
            outv[4 + 2 * l, sl] = t1[0]

    def idx_pass(l):
        res = float(RES[l])
        lshift = np.uint32(l * T)
        wxv, wyv, wzv = wbufs[l % 2]
        idxv = idxbufs[l % 2]

        @plsc.parallel_loop(0, NV)
        def idx_body(v):
            sl = pl.ds(v * 16, 16)
            px = xv[sl] * res
            py = yv[sl] * res
            pz = zv[sl] * res
            ix = px.astype(jnp.int32)
            iy = py.astype(jnp.int32)
            iz = pz.astype(jnp.int32)
            wxv[sl] = px - ix.astype(jnp.float32)
            wyv[sl] = py - iy.astype(jnp.float32)
            wzv[sl] = pz - iz.astype(jnp.float32)
            hx0 = ix.astype(jnp.uint32)
            hy0 = iy.astype(jnp.uint32) * P1
            hz0 = iz.astype(jnp.uint32) * P2
            hx = (hx0, hx0 + jnp.uint32(1))
            hy = (hy0, hy0 + P1)
            hz = (hz0, hz0 + P2)
            for c, (cx, cy, cz) in enumerate(_corners()):
                h = ((hx[cx] ^ hy[cy] ^ hz[cz]) & MASK) | lshift
                idxv[pl.ds(v * 128 + c * 16, 16)] = h.astype(jnp.int32)

    def acc_pass(l):
        wxv, wyv, wzv = wbufs[l % 2]
        rowsv = rowbufs[l % 2]

        @plsc.parallel_loop(0, NV)
        def acc_body(v):
            sl = pl.ds(v * 16, 16)
            wx = wxv[sl]
            wy = wyv[sl]
            wz = wzv[sl]
            cwx = (1.0 - wx, wx)
            cwy = (1.0 - wy, wy)
            cwz = (1.0 - wz, wz)
            zc = jnp.zeros((16,), jnp.int32)
            oc = jnp.ones((16,), jnp.int32)
            t0 = []
            t1 = []
            for c, (cx, cy, cz) in enumerate(_corners()):
                wt = cwx[cx] * cwy[cy] * cwz[cz]
                col = v * 128 + c * 16 + iota
                g0 = plsc.load_gather(rowsv, [col, zc])
                g1 = plsc.load_gather(rowsv, [col, oc])
                t0.append(wt * g0)
                t1.append(wt * g1)
            while len(t0) > 1:
                t0 = [a + b for a, b in zip(t0[::2], t0[1::2])]
                t1 = [a + b for a, b in zip(t1[::2], t1[1::2])]
            outv[3 + 2 * l, sl] = t0[0]
            outv[4 + 2 * l, sl] = t1[0]

    def gather_start(l):
        idxv = idxbufs[l % 2]
        rowsv = rowbufs[l % 2]
        sem = sems[l % 2]
        for j in range(KSUB):
            sl = pl.ds(j * B2, B2)
            pltpu.make_async_copy(tbl.at[idxv.at[sl]], rowsv.at[sl, :],
                                  sem).start()

    def gather_drain(l):
        idxv = idxbufs[l % 2]
        rowsv = rowbufs[l % 2]
        sem = sems[l % 2]
        for j in range(KSUB):
            sl = pl.ds(j * B2, B2)
            pltpu.make_async_copy(tbl.at[idxv.at[sl]], rowsv.at[sl, :],
                                  sem).wait()

    def chunk_body(ch, _):
        p0 = base + ch * C
        pltpu.sync_copy(in_flat.at[pl.ds(p0 * 3, 3 * C)], inbuf)

        @plsc.parallel_loop(0, NV)
        def prep_body(v):
            sl = pl.ds(v * 16, 16)
            i3 = (v * 16 + iota) * 3
            rx = plsc.load_gather(inbuf, [i3])
            ry = plsc.load_gather(inbuf, [i3 + 1])
            rz = plsc.load_gather(inbuf, [i3 + 2])
            outv[0, sl] = rx
            outv[1, sl] = ry
            outv[2, sl] = rz
            xv[sl] = (rx - off0) * sc0
            yv[sl] = (ry - off1) * sc1
            zv[sl] = (rz - off2) * sc2

        idx_pass(2)
        gather_start(2)
        lut_pass(0)
        lut_pass(1)
        for l in range(3, N_LEVELS):
            idx_pass(l)
            gather_start(l)
            gather_drain(l - 1)
            acc_pass(l - 1)
        gather_drain(N_LEVELS - 1)
        acc_pass(N_LEVELS - 1)

        @plsc.parallel_loop(0, NSB)
        def relayout_body(s):
            pbase = s * 16
            for j in range(D_OUT):
                colv = rtabv[pl.ds(j * 16, 16)]
                pointv = rtabv[pl.ds(D_OUT * 16 + j * 16, 16)] + pbase
                g = plsc.load_gather(outv, [colv, pointv])
                outiv[pl.ds(s * (16 * D_OUT) + j * 16, 16)] = g

        pltpu.sync_copy(outiv, out.at[pl.ds(p0 * D_OUT, C * D_OUT)])
        return 0

    lax.fori_loop(0, nch, chunk_body, 0)


def kernel(input, bound_min, bound_max, table):
    n = input.shape[0]
    in_flat = input.reshape(-1)
    scale = 1.0 / (bound_max - bound_min)
    prm = jnp.concatenate(
        [jnp.repeat(bound_min, 16), jnp.repeat(scale, 16)]
    ).astype(jnp.float32)
    tbl = table.reshape(N_LEVELS * T, F)
    rtab = jnp.asarray(_RELAYOUT_TAB)
    dhtab = jnp.asarray(_DHTAB)

    mesh = plsc.VectorSubcoreMesh(core_axis_name="c", subcore_axis_name="s")
    run = pl.kernel(
        functools.partial(_body, n),
        out_type=jax.ShapeDtypeStruct((n * D_OUT,), jnp.float32),
        mesh=mesh,
        scratch_types=[
            pltpu.VMEM((2 * D_OUT * 16,), jnp.int32),
            pltpu.VMEM((96,), jnp.float32),
            pltpu.VMEM((3 * C,), jnp.float32),
            pltpu.VMEM((C,), jnp.float32),
            pltpu.VMEM((C,), jnp.float32),
            pltpu.VMEM((C,), jnp.float32),
            pltpu.VMEM((C,), jnp.float32),
            pltpu.VMEM((C,), jnp.float32),
            pltpu.VMEM((C,), jnp.float32),
            pltpu.VMEM((C,), jnp.float32),
            pltpu.VMEM((C,), jnp.float32),
            pltpu.VMEM((C,), jnp.float32),
            pltpu.VMEM((8 * C,), jnp.int32),
            pltpu.VMEM((8 * C,), jnp.int32),
            pltpu.VMEM((8 * C, F), jnp.float32),
            pltpu.VMEM((8 * C, F), jnp.float32),
            pltpu.VMEM((D_OUT, C), jnp.float32),
            pltpu.VMEM((C * D_OUT,), jnp.float32),
            pltpu.VMEM((_lut_pad(LUT_K[0]),), jnp.float32),
            pltpu.VMEM((_lut_pad(LUT_K[0]),), jnp.float32),
            pltpu.VMEM((_lut_pad(LUT_K[1]),), jnp.float32),
            pltpu.VMEM((_lut_pad(LUT_K[1]),), jnp.float32),
            pltpu.SemaphoreType.DMA,
            pltpu.SemaphoreType.DMA,
        ],
        compiler_params=pltpu.CompilerParams(
            needs_layout_passes=False, use_tc_tiling_on_sc=False),
    )
    return run(in_flat, prm, tbl, rtab, dhtab).reshape(n, D_OUT)
